# baseline (speedup 1.0000x reference)
import functools

import jax
import jax.numpy as jnp
from jax import lax
from jax.experimental import pallas as pl
from jax.experimental.pallas import tpu as pltpu
from jax.experimental.pallas import tpu_sc as plsc

NC = 2
NS = 16
NW = NC * NS
B = 128
D_H = 64

_MESH = plsc.VectorSubcoreMesh(
    core_axis_name="c", subcore_axis_name="s", num_cores=NC, num_subcores=NS)
_SC_PARAMS = pltpu.CompilerParams(
    needs_layout_passes=False, use_tc_tiling_on_sc=False,
    disable_bounds_checks=True)


def _zero_rows(buf, nrows, ncols):
    def body(r, _):
        for q in range(ncols // 16):
            buf[r, pl.ds(q * 16, 16)] = jnp.zeros((16,), jnp.float32)
        return 0
    lax.fori_loop(0, nrows, body, 0)


def _deg_body(nch, n_pad, colp, ewp, degp, colb, ewb, zb, shared):
    c = lax.axis_index("c")
    s = lax.axis_index("s")
    wid = c * NS + s
    stripe = n_pad // NS
    pltpu.sync_copy(colp.at[wid], colb)
    pltpu.sync_copy(ewp.at[wid], ewb)
    def zbody(k, _):
        zb[pl.ds(k * 16, 16)] = jnp.zeros((16,), jnp.float32)
        return 0
    lax.fori_loop(0, stripe // 16, zbody, 0)
    pltpu.sync_copy(zb, shared.at[pl.ds(s * stripe, stripe)])
    plsc.subcore_barrier()
    def chunk(j, _):
        pltpu.sync_copy(ewb.at[j], shared.at[colb.at[j]], add=True)
        return 0
    lax.fori_loop(0, nch, chunk, 0)
    plsc.subcore_barrier()
    pltpu.sync_copy(shared.at[pl.ds(s * stripe, stripe)],
                    degp.at[c, pl.ds(s * stripe, stripe)])


def _sc_degree(colp, ewp, n_pad):
    nch = colp.shape[1]
    body = functools.partial(_deg_body, nch, n_pad)
    f = pl.kernel(
        body,
        out_type=jax.ShapeDtypeStruct((NC, n_pad), jnp.float32),
        mesh=_MESH,
        scratch_types=[
            pltpu.VMEM((nch, B), jnp.int32),
            pltpu.VMEM((nch, B), jnp.float32),
            pltpu.VMEM((n_pad // NS,), jnp.float32),
            pltpu.VMEM_SHARED((n_pad,), jnp.float32),
        ],
        compiler_params=_SC_PARAMS,
    )
    return f(colp, ewp)


def _msg_body(nch, n_pad, compute_norm, *args):
    if compute_norm:
        (rowp, colp, ewp, dinvh, xwh, parts, nrmout,
         rowb, colb, nrmb, dinvb, b0, b1, b2, b3, f0, f1, f2, f3,
         shared, g0, g1, g2, g3, s0, s1, s2, s3) = args
    else:
        (rowp, colp, nrmp, xwh, parts,
         rowb, colb, nrmb, b0, b1, b2, b3, f0, f1, f2, f3,
         shared, g0, g1, g2, g3, s0, s1, s2, s3) = args
    bufs = (b0, b1, b2, b3)
    sbufs = (f0, f1, f2, f3)
    gsems = (g0, g1, g2, g3)
    ssems = (s0, s1, s2, s3)
    c = lax.axis_index("c")
    s = lax.axis_index("s")
    wid = c * NS + s
    stripe = n_pad // NS
    pltpu.sync_copy(rowp.at[wid], rowb)
    pltpu.sync_copy(colp.at[wid], colb)
    if compute_norm:
        pltpu.sync_copy(ewp.at[wid], nrmb)
        pltpu.sync_copy(dinvh, dinvb)
        @plsc.parallel_loop(0, nch, 1, unroll=2)
        def _(j):
            for i in range(B // 16):
                sl = pl.ds(j * B + i * 16, 16)
                nv = (plsc.load_gather(dinvb, [rowb[j, pl.ds(i * 16, 16)]])
                      * nrmb[sl]
                      * plsc.load_gather(dinvb, [colb[j, pl.ds(i * 16, 16)]]))
                nrmb[sl] = nv
        pltpu.sync_copy(nrmb, nrmout.at[wid])
    else:
        pltpu.sync_copy(nrmp.at[wid], nrmb)
    _zero_rows(f0, B, D_H)
    for k in range(stripe // B):
        pltpu.sync_copy(f0, shared.at[pl.ds(s * stripe + k * B, B)])
    plsc.subcore_barrier()

    def scale(buf, sbuf, j):
        jb = j * B
        @plsc.parallel_loop(0, B // 16, 1)
        def _(t):
            nv16 = nrmb[pl.ds(jb + t * 16, 16)]
            for u in range(16):
                e = t * 16 + u
                sv = jnp.full((16,), nv16[u], jnp.float32)
                for h in range(D_H // 32):
                    w = plsc.bitcast(buf[e, pl.ds(h * 32, 32)], jnp.int32)
                    flo = plsc.bitcast(
                        lax.shift_left(w, jnp.int32(16)), jnp.float32)
                    fhi = plsc.bitcast(
                        w & jnp.int32(-65536), jnp.float32)
                    sbuf[e, pl.ds(h * 32, 16)] = flo * sv
                    sbuf[e, pl.ds(h * 32 + 16, 16)] = fhi * sv

    pltpu.async_copy(xwh.at[rowb.at[0]], bufs[0], gsems[0])
    pltpu.async_copy(xwh.at[rowb.at[1]], bufs[1], gsems[1])

    def body(g, _):
        for u in range(4):
            j = 4 * g + u
            bu, fu, gu, su = bufs[u], sbufs[u], gsems[u], ssems[u]
            u2 = (u + 2) % 4
            pltpu.make_async_copy(xwh.at[rowb.at[j]], bu, gu).wait()
            scale(bu, fu, j)
            pltpu.async_copy(fu, shared.at[colb.at[j]], su, add=True)

            @pl.when(j >= 2)
            def _():
                pltpu.make_async_copy(
                    sbufs[u2], shared.at[colb.at[j]], ssems[u2]).wait()

            @pl.when(j + 2 < nch)
            def _():
                pltpu.async_copy(
                    xwh.at[rowb.at[j + 2]], bufs[u2], gsems[u2])
        return 0
    lax.fori_loop(0, nch // 4, body, 0)
    for jt in (nch - 2, nch - 1):
        pltpu.make_async_copy(
            sbufs[jt % 4], shared.at[colb.at[0]], ssems[jt % 4]).wait()
    plsc.subcore_barrier()
    pltpu.sync_copy(shared.at[pl.ds(s * stripe, stripe)],
                    parts.at[c, pl.ds(s * stripe, stripe)])


def _sc_layer1(rowp, colp, ewp, dinv_flat, xw, n_pad):
    nch = rowp.shape[1]
    body = functools.partial(_msg_body, nch, n_pad, True)
    f = pl.kernel(
        body,
        out_type=(jax.ShapeDtypeStruct((NC, n_pad, D_H), jnp.float32),
                  jax.ShapeDtypeStruct((NW, nch * B), jnp.float32)),
        mesh=_MESH,
        scratch_types=[
            pltpu.VMEM((nch, B), jnp.int32),
            pltpu.VMEM((nch, B), jnp.int32),
            pltpu.VMEM((nch * B,), jnp.float32),
            pltpu.VMEM((n_pad,), jnp.float32),
        ] + [pltpu.VMEM((B, D_H), jnp.bfloat16)] * 4
          + [pltpu.VMEM((B, D_H), jnp.float32)] * 4 + [
            pltpu.VMEM_SHARED((n_pad, D_H), jnp.float32),
        ] + [pltpu.SemaphoreType.DMA] * 8,
        compiler_params=_SC_PARAMS,
    )
    return f(rowp, colp, ewp, dinv_flat, xw)


def _sc_layer2(rowp, colp, nrmp, xw, n_pad):
    nch = rowp.shape[1]
    body = functools.partial(_msg_body, nch, n_pad, False)
    f = pl.kernel(
        body,
        out_type=jax.ShapeDtypeStruct((NC, n_pad, D_H), jnp.float32),
        mesh=_MESH,
        scratch_types=[
            pltpu.VMEM((nch, B), jnp.int32),
            pltpu.VMEM((nch, B), jnp.int32),
            pltpu.VMEM((nch * B,), jnp.float32),
        ] + [pltpu.VMEM((B, D_H), jnp.bfloat16)] * 4
          + [pltpu.VMEM((B, D_H), jnp.float32)] * 4 + [
            pltpu.VMEM_SHARED((n_pad, D_H), jnp.float32),
        ] + [pltpu.SemaphoreType.DMA] * 8,
        compiler_params=_SC_PARAMS,
    )
    return f(rowp, colp, nrmp, xw)


def _tc1_body(dp_ref, xp_ref, w_ref, dinv_ref, ss_ref, xw_ref):
    dp = dp_ref[...]
    deg = dp[0] + dp[1] + 1.0
    dinv = jnp.where(deg > 0, lax.rsqrt(deg), 0.0)
    dinv_ref[...] = dinv
    ss_ref[...] = dinv * dinv
    xw_ref[...] = jnp.dot(xp_ref[...], w_ref[...],
                          preferred_element_type=jnp.float32)


def _tc2_body(parts_ref, xw_ref, ss_ref, b_ref, w_ref, xw2_ref):
    p = parts_ref[...]
    h = p[0] + p[1] + xw_ref[...] * ss_ref[...] + b_ref[...]
    h = jnp.maximum(h, 0.0)
    xw2_ref[...] = jnp.dot(h, w_ref[...], preferred_element_type=jnp.float32)


def _tc3_body(parts_ref, xw_ref, ss_ref, b_ref, wc_ref, bc_ref, out_ref):
    p = parts_ref[...]
    h = p[0] + p[1] + xw_ref[...] * ss_ref[...] + b_ref[...]
    h = jnp.maximum(h, 0.0)
    out_ref[...] = (jnp.dot(h, wc_ref[...], preferred_element_type=jnp.float32)
                    + bc_ref[...])


def _pack_cols(a):
    n_, d = a.shape
    b = a.reshape(n_, d // 32, 2, 16).astype(jnp.bfloat16)
    return b.transpose(0, 1, 3, 2).reshape(n_, d)


def kernel(x, edge_index, edge_attr, W1, b1, W2, b2, Wc, bc):
    n, d_in = x.shape
    e = edge_attr.shape[0]
    n_cls = Wc.shape[1]

    n_pad = ((n + NS * B - 1) // (NS * B)) * (NS * B)
    nch = (e + NW * B - 1) // (NW * B)
    nch = ((nch + 3) // 4) * 4
    e_pad = NW * nch * B
    row = edge_index[0]
    col = edge_index[1]
    zpad_i = jnp.zeros((e_pad - e,), jnp.int32)
    rowp = jnp.concatenate([row, zpad_i]).reshape(NW, nch, B)
    colp = jnp.concatenate([col, zpad_i]).reshape(NW, nch, B)
    ewp = jnp.concatenate(
        [edge_attr, jnp.zeros((e_pad - e,), jnp.float32)]).reshape(NW, nch * B)
    xp = jnp.pad(x, ((0, n_pad - n), (0, 0)))
    b1r = b1.reshape(1, D_H)
    b2r = b2.reshape(1, D_H)
    bcr = bc.reshape(1, n_cls)

    degp = _sc_degree(colp, ewp.reshape(NW, nch, B), n_pad)

    dinv2, ss2, xw1 = pl.pallas_call(
        _tc1_body,
        out_shape=(jax.ShapeDtypeStruct((n_pad // 128, 128), jnp.float32),
                   jax.ShapeDtypeStruct((n_pad // 128, 128), jnp.float32),
                   jax.ShapeDtypeStruct((n_pad, D_H), jnp.float32)),
    )(degp.reshape(NC, n_pad // 128, 128), xp, W1)
    dinv_flat = dinv2.reshape(n_pad)
    ss_col = ss2.reshape(n_pad, 1)

    parts1, nrmp = _sc_layer1(rowp, colp, ewp, dinv_flat,
                              _pack_cols(xw1), n_pad)

    xw2 = pl.pallas_call(
        _tc2_body,
        out_shape=jax.ShapeDtypeStruct((n_pad, D_H), jnp.float32),
    )(parts1, xw1, ss_col, b1r, W2)

    parts2 = _sc_layer2(rowp, colp, nrmp, _pack_cols(xw2), n_pad)

    out = pl.pallas_call(
        _tc3_body,
        out_shape=jax.ShapeDtypeStruct((n_pad, n_cls), jnp.float32),
    )(parts2, xw2, ss_col, b2r, Wc, bcr)

    return out[:n]


# speedup vs baseline: 1.0124x; 1.0124x over previous
import functools

import jax
import jax.numpy as jnp
from jax import lax
from jax.experimental import pallas as pl
from jax.experimental.pallas import tpu as pltpu
from jax.experimental.pallas import tpu_sc as plsc

NC = 2
NS = 16
NW = NC * NS
B = 128
D_H = 64

_MESH = plsc.VectorSubcoreMesh(
    core_axis_name="c", subcore_axis_name="s", num_cores=NC, num_subcores=NS)
_SC_PARAMS = pltpu.CompilerParams(
    needs_layout_passes=False, use_tc_tiling_on_sc=False,
    disable_bounds_checks=True)


def _zero_rows(buf, nrows, ncols):
    def body(r, _):
        for q in range(ncols // 16):
            buf[r, pl.ds(q * 16, 16)] = jnp.zeros((16,), jnp.float32)
        return 0
    lax.fori_loop(0, nrows, body, 0)


def _deg_body(nch, n_pad, colp, ewp, degp, colb, ewb, zb, shared):
    c = lax.axis_index("c")
    s = lax.axis_index("s")
    wid = c * NS + s
    stripe = n_pad // NS
    pltpu.sync_copy(colp.at[wid], colb)
    pltpu.sync_copy(ewp.at[wid], ewb)
    def zbody(k, _):
        zb[pl.ds(k * 16, 16)] = jnp.zeros((16,), jnp.float32)
        return 0
    lax.fori_loop(0, stripe // 16, zbody, 0)
    pltpu.sync_copy(zb, shared.at[pl.ds(s * stripe, stripe)])
    plsc.subcore_barrier()
    def chunk(j, _):
        pltpu.sync_copy(ewb.at[j], shared.at[colb.at[j]], add=True)
        return 0
    lax.fori_loop(0, nch, chunk, 0)
    plsc.subcore_barrier()
    pltpu.sync_copy(shared.at[pl.ds(s * stripe, stripe)],
                    degp.at[c, pl.ds(s * stripe, stripe)])


def _sc_degree(colp, ewp, n_pad):
    nch = colp.shape[1]
    body = functools.partial(_deg_body, nch, n_pad)
    f = pl.kernel(
        body,
        out_type=jax.ShapeDtypeStruct((NC, n_pad), jnp.float32),
        mesh=_MESH,
        scratch_types=[
            pltpu.VMEM((nch, B), jnp.int32),
            pltpu.VMEM((nch, B), jnp.float32),
            pltpu.VMEM((n_pad // NS,), jnp.float32),
            pltpu.VMEM_SHARED((n_pad,), jnp.float32),
        ],
        compiler_params=_SC_PARAMS,
    )
    return f(colp, ewp)


def _msg_body(nch, n_pad, compute_norm, *args):
    if compute_norm:
        (rowp, colp, ewp, dinvh, xwh, parts, nrmout,
         rowb, colb, nrmb, dinvb, b0, b1, b2, b3, f0, f1, f2, f3,
         shared, g0, g1, g2, g3, s0, s1, s2, s3) = args
    else:
        (rowp, colp, nrmp, xwh, parts,
         rowb, colb, nrmb, b0, b1, b2, b3, f0, f1, f2, f3,
         shared, g0, g1, g2, g3, s0, s1, s2, s3) = args
    bufs = (b0, b1, b2, b3)
    sbufs = (f0, f1, f2, f3)
    gsems = (g0, g1, g2, g3)
    ssems = (s0, s1, s2, s3)
    c = lax.axis_index("c")
    s = lax.axis_index("s")
    wid = c * NS + s
    stripe = n_pad // NS
    pltpu.sync_copy(rowp.at[wid], rowb)
    pltpu.sync_copy(colp.at[wid], colb)
    if compute_norm:
        pltpu.sync_copy(ewp.at[wid], nrmb)
        pltpu.sync_copy(dinvh, dinvb)
        @plsc.parallel_loop(0, nch, 1, unroll=2)
        def _(j):
            for i in range(B // 16):
                sl = pl.ds(j * B + i * 16, 16)
                nv = (plsc.load_gather(dinvb, [rowb[j, pl.ds(i * 16, 16)]])
                      * nrmb[sl]
                      * plsc.load_gather(dinvb, [colb[j, pl.ds(i * 16, 16)]]))
                nrmb[sl] = nv
        pltpu.sync_copy(nrmb, nrmout.at[wid])
    else:
        pltpu.sync_copy(nrmp.at[wid], nrmb)
    _zero_rows(f0, B, D_H)
    for k in range(stripe // B):
        pltpu.sync_copy(f0, shared.at[pl.ds(s * stripe + k * B, B)])
    plsc.subcore_barrier()

    def scale(buf, sbuf, j):
        jb = j * B
        @plsc.parallel_loop(0, B // 16, 1)
        def _(t):
            nv16 = nrmb[pl.ds(jb + t * 16, 16)]
            for u in range(16):
                e = t * 16 + u
                sv = jnp.full((16,), nv16[u], jnp.float32)
                for h in range(D_H // 32):
                    w = plsc.bitcast(buf[e, pl.ds(h * 32, 32)], jnp.int32)
                    flo = plsc.bitcast(
                        lax.shift_left(w, jnp.int32(16)), jnp.float32)
                    fhi = plsc.bitcast(
                        w & jnp.int32(-65536), jnp.float32)
                    sbuf[e, pl.ds(h * 32, 16)] = flo * sv
                    sbuf[e, pl.ds(h * 32 + 16, 16)] = fhi * sv

    pltpu.async_copy(xwh.at[rowb.at[0]], bufs[0], gsems[0])
    pltpu.async_copy(xwh.at[rowb.at[1]], bufs[1], gsems[1])
    pltpu.async_copy(xwh.at[rowb.at[2]], bufs[2], gsems[2])

    def body(g, _):
        for u in range(4):
            j = 4 * g + u
            bu, fu, gu, su = bufs[u], sbufs[u], gsems[u], ssems[u]
            u2 = (u + 2) % 4
            u3 = (u + 3) % 4
            pltpu.make_async_copy(xwh.at[rowb.at[j]], bu, gu).wait()
            scale(bu, fu, j)
            pltpu.async_copy(fu, shared.at[colb.at[j]], su, add=True)

            @pl.when(j >= 2)
            def _():
                pltpu.make_async_copy(
                    sbufs[u2], shared.at[colb.at[j]], ssems[u2]).wait()

            @pl.when(j + 3 < nch)
            def _():
                pltpu.async_copy(
                    xwh.at[rowb.at[j + 3]], bufs[u3], gsems[u3])
        return 0
    lax.fori_loop(0, nch // 4, body, 0)
    for jt in (nch - 2, nch - 1):
        pltpu.make_async_copy(
            sbufs[jt % 4], shared.at[colb.at[0]], ssems[jt % 4]).wait()
    plsc.subcore_barrier()
    pltpu.sync_copy(shared.at[pl.ds(s * stripe, stripe)],
                    parts.at[c, pl.ds(s * stripe, stripe)])


def _sc_layer1(rowp, colp, ewp, dinv_flat, xw, n_pad):
    nch = rowp.shape[1]
    body = functools.partial(_msg_body, nch, n_pad, True)
    f = pl.kernel(
        body,
        out_type=(jax.ShapeDtypeStruct((NC, n_pad, D_H), jnp.float32),
                  jax.ShapeDtypeStruct((NW, nch * B), jnp.float32)),
        mesh=_MESH,
        scratch_types=[
            pltpu.VMEM((nch, B), jnp.int32),
            pltpu.VMEM((nch, B), jnp.int32),
            pltpu.VMEM((nch * B,), jnp.float32),
            pltpu.VMEM((n_pad,), jnp.float32),
        ] + [pltpu.VMEM((B, D_H), jnp.bfloat16)] * 4
          + [pltpu.VMEM((B, D_H), jnp.float32)] * 4 + [
            pltpu.VMEM_SHARED((n_pad, D_H), jnp.float32),
        ] + [pltpu.SemaphoreType.DMA] * 8,
        compiler_params=_SC_PARAMS,
    )
    return f(rowp, colp, ewp, dinv_flat, xw)


def _sc_layer2(rowp, colp, nrmp, xw, n_pad):
    nch = rowp.shape[1]
    body = functools.partial(_msg_body, nch, n_pad, False)
    f = pl.kernel(
        body,
        out_type=jax.ShapeDtypeStruct((NC, n_pad, D_H), jnp.float32),
        mesh=_MESH,
        scratch_types=[
            pltpu.VMEM((nch, B), jnp.int32),
            pltpu.VMEM((nch, B), jnp.int32),
            pltpu.VMEM((nch * B,), jnp.float32),
        ] + [pltpu.VMEM((B, D_H), jnp.bfloat16)] * 4
          + [pltpu.VMEM((B, D_H), jnp.float32)] * 4 + [
            pltpu.VMEM_SHARED((n_pad, D_H), jnp.float32),
        ] + [pltpu.SemaphoreType.DMA] * 8,
        compiler_params=_SC_PARAMS,
    )
    return f(rowp, colp, nrmp, xw)


def _tc1_body(dp_ref, xp_ref, w_ref, dinv_ref, ss_ref, xw_ref):
    dp = dp_ref[...]
    deg = dp[0] + dp[1] + 1.0
    dinv = jnp.where(deg > 0, lax.rsqrt(deg), 0.0)
    dinv_ref[...] = dinv
    ss_ref[...] = dinv * dinv
    xw_ref[...] = jnp.dot(xp_ref[...], w_ref[...],
                          preferred_element_type=jnp.float32)


def _tc2_body(parts_ref, xw_ref, ss_ref, b_ref, w_ref, xw2_ref):
    p = parts_ref[...]
    h = p[0] + p[1] + xw_ref[...] * ss_ref[...] + b_ref[...]
    h = jnp.maximum(h, 0.0)
    xw2_ref[...] = jnp.dot(h, w_ref[...], preferred_element_type=jnp.float32)


def _tc3_body(parts_ref, xw_ref, ss_ref, b_ref, wc_ref, bc_ref, out_ref):
    p = parts_ref[...]
    h = p[0] + p[1] + xw_ref[...] * ss_ref[...] + b_ref[...]
    h = jnp.maximum(h, 0.0)
    out_ref[...] = (jnp.dot(h, wc_ref[...], preferred_element_type=jnp.float32)
                    + bc_ref[...])


def _pack_cols(a):
    n_, d = a.shape
    b = a.reshape(n_, d // 32, 2, 16).astype(jnp.bfloat16)
    return b.transpose(0, 1, 3, 2).reshape(n_, d)


def kernel(x, edge_index, edge_attr, W1, b1, W2, b2, Wc, bc):
    n, d_in = x.shape
    e = edge_attr.shape[0]
    n_cls = Wc.shape[1]

    n_pad = ((n + NS * B - 1) // (NS * B)) * (NS * B)
    nch = (e + NW * B - 1) // (NW * B)
    nch = ((nch + 3) // 4) * 4
    e_pad = NW * nch * B
    row = edge_index[0]
    col = edge_index[1]
    zpad_i = jnp.zeros((e_pad - e,), jnp.int32)
    rowp = jnp.concatenate([row, zpad_i]).reshape(NW, nch, B)
    colp = jnp.concatenate([col, zpad_i]).reshape(NW, nch, B)
    ewp = jnp.concatenate(
        [edge_attr, jnp.zeros((e_pad - e,), jnp.float32)]).reshape(NW, nch * B)
    xp = jnp.pad(x, ((0, n_pad - n), (0, 0)))
    b1r = b1.reshape(1, D_H)
    b2r = b2.reshape(1, D_H)
    bcr = bc.reshape(1, n_cls)

    degp = _sc_degree(colp, ewp.reshape(NW, nch, B), n_pad)

    dinv2, ss2, xw1 = pl.pallas_call(
        _tc1_body,
        out_shape=(jax.ShapeDtypeStruct((n_pad // 128, 128), jnp.float32),
                   jax.ShapeDtypeStruct((n_pad // 128, 128), jnp.float32),
                   jax.ShapeDtypeStruct((n_pad, D_H), jnp.float32)),
    )(degp.reshape(NC, n_pad // 128, 128), xp, W1)
    dinv_flat = dinv2.reshape(n_pad)
    ss_col = ss2.reshape(n_pad, 1)

    parts1, nrmp = _sc_layer1(rowp, colp, ewp, dinv_flat,
                              _pack_cols(xw1), n_pad)

    xw2 = pl.pallas_call(
        _tc2_body,
        out_shape=jax.ShapeDtypeStruct((n_pad, D_H), jnp.float32),
    )(parts1, xw1, ss_col, b1r, W2)

    parts2 = _sc_layer2(rowp, colp, nrmp, _pack_cols(xw2), n_pad)

    out = pl.pallas_call(
        _tc3_body,
        out_shape=jax.ShapeDtypeStruct((n_pad, n_cls), jnp.float32),
    )(parts2, xw2, ss_col, b2r, Wc, bcr)

    return out[:n]
